# baseline (speedup 1.0000x reference)
import functools

import numpy as np
import jax
import jax.numpy as jnp
from jax import lax
from jax.experimental import pallas as pl
from jax.experimental.pallas import tpu as pltpu
from jax.experimental.pallas import tpu_sc as plsc

EMBED = 32
MAX_LEN = 202
SEQ = 200
BATCH = 4096
N_TOK = BATCH * SEQ
NUM_WORKERS = 32
PER_W = N_TOK // NUM_WORKERS
CHUNK = 512
N_CHUNKS = PER_W // CHUNK
KSUB = CHUNK // 128
IDX_ROWS = N_TOK // 128
PIDX_ROWS = PER_W // 128


def _make_pe() -> jnp.ndarray:
    pos = np.arange(MAX_LEN, dtype=np.float32)[:, None]
    div = np.exp(np.arange(0, EMBED, 2, dtype=np.float32) * -(np.log(10000.0) / EMBED))
    pe = np.zeros((MAX_LEN, EMBED), dtype=np.float32)
    pe[:, 0::2] = np.sin(pos * div)
    pe[:, 1::2] = np.cos(pos * div)
    return jnp.asarray(pe[:SEQ])


def _make_pidx() -> jnp.ndarray:
    return jnp.asarray((np.arange(PER_W, dtype=np.int32) % SEQ).reshape(PIDX_ROWS, 128))


_MESH = plsc.VectorSubcoreMesh(core_axis_name="c", subcore_axis_name="s")


@functools.partial(
    pl.kernel,
    out_type=jax.ShapeDtypeStruct((N_TOK, EMBED), jnp.float32),
    mesh=_MESH,
    compiler_params=pltpu.CompilerParams(use_tc_tiling_on_sc=False),
    scratch_types=[
        pltpu.VMEM((KSUB, 128), jnp.int32),
        pltpu.VMEM((KSUB, 128), jnp.int32),
        pltpu.VMEM((KSUB, 128), jnp.int32),
        pltpu.VMEM((KSUB, 128), jnp.int32),
        pltpu.VMEM((KSUB, 128), jnp.int32),
        pltpu.VMEM((CHUNK, EMBED), jnp.float32),
        pltpu.SemaphoreType.DMA,
        pltpu.SemaphoreType.DMA,
        pltpu.SemaphoreType.DMA,
    ],
)
def _emb_kernel(grid_tab, time_tab, event_tab, hand_tab, pe_tab,
                gidx, tidx, eidx, hidx, pidx, out,
                s_gi, s_ti, s_ei, s_hi, s_pi, r_acc,
                sem_i, sem_g, sem_a):
    wid = lax.axis_index("s") * 2 + lax.axis_index("c")
    idx_row0 = wid * (PER_W // 128)

    def chunk_body(i, carry):
        rbase = idx_row0 + i * KSUB
        pbase = lax.rem(i * KSUB, PIDX_ROWS)
        cps = [
            pltpu.async_copy(gidx.at[pl.ds(rbase, KSUB)], s_gi, sem_i),
            pltpu.async_copy(tidx.at[pl.ds(rbase, KSUB)], s_ti, sem_i),
            pltpu.async_copy(eidx.at[pl.ds(rbase, KSUB)], s_ei, sem_i),
            pltpu.async_copy(hidx.at[pl.ds(rbase, KSUB)], s_hi, sem_i),
            pltpu.async_copy(pidx.at[pl.ds(pbase, KSUB)], s_pi, sem_i),
        ]
        for cp in cps:
            cp.wait()
        gcps = []
        for k in range(KSUB):
            dst = pl.ds(k * 128, 128)
            gcps.append(pltpu.async_copy(grid_tab.at[s_gi.at[k]], r_acc.at[dst], sem_g))
        for cp in gcps:
            cp.wait()
        acps = []
        for k in range(KSUB):
            dst = pl.ds(k * 128, 128)
            acps.append(pltpu.async_copy(time_tab.at[s_ti.at[k]], r_acc.at[dst], sem_a, add=True))
            acps.append(pltpu.async_copy(event_tab.at[s_ei.at[k]], r_acc.at[dst], sem_a, add=True))
            acps.append(pltpu.async_copy(hand_tab.at[s_hi.at[k]], r_acc.at[dst], sem_a, add=True))
            acps.append(pltpu.async_copy(pe_tab.at[s_pi.at[k]], r_acc.at[dst], sem_a, add=True))
        for cp in acps:
            cp.wait()

        base_tok = wid * PER_W + i * CHUNK
        pltpu.sync_copy(r_acc, out.at[pl.ds(base_tok, CHUNK)])
        return carry

    lax.fori_loop(0, N_CHUNKS, chunk_body, 0)


def kernel(grid, timestamp, event, hand, grid_table, time_table, event_table,
           hand_table, train_mode):
    pe = _make_pe()
    pidx = _make_pidx()
    gi = grid.astype(jnp.int32).reshape(IDX_ROWS, 128)
    ti = timestamp.astype(jnp.int32).reshape(IDX_ROWS, 128)
    ei = event.astype(jnp.int32).reshape(IDX_ROWS, 128)
    hi = hand.astype(jnp.int32).reshape(IDX_ROWS, 128)
    out = _emb_kernel(grid_table, time_table, event_table, hand_table, pe,
                      gi, ti, ei, hi, pidx)
    return out.reshape(BATCH, SEQ, EMBED)
